# baseline (speedup 1.0000x reference)
import functools

import jax
import jax.numpy as jnp
from jax import lax
from jax.experimental import pallas as pl
from jax.experimental.pallas import tpu as pltpu
from jax.experimental.pallas import tpu_sc as plsc

N_NODES = 10000
N_EDGES = 320000
DEGREE = 32
NODE_SIZE = 128
EDGE_SIZE = 16
OUT_SIZE = 128
EPS = 1e-5

NUM_WORKERS = 32
CHUNK = 4
IDX_PER_CHUNK = CHUNK * DEGREE
NUM_CHUNKS = N_NODES // CHUNK
NUM_PAIRS = NUM_CHUNKS // 2
IDX_PAD_ROWS = 2504
LANES = 16
NVEC = NODE_SIZE // LANES
NBUF = 2

MAX_CPW_N = 80
MAX_CPW_E = 79


def _gather_sum_body(table_hbm, idx_hbm, out_hbm, idx_v, rows, out_v, sems,
                     width, aligned):
    nv = width // LANES
    max_cpw = MAX_CPW_N if aligned else MAX_CPW_E
    cid = lax.axis_index("c")
    sid = lax.axis_index("s")
    wid = sid * 2 + cid

    if aligned:
        p0 = wid * NUM_PAIRS // NUM_WORKERS
        p1 = (wid + 1) * NUM_PAIRS // NUM_WORKERS
        base = 2 * p0
        cnt = 2 * (p1 - p0)
        base_al = (base // 8) * 8
        off = base - base_al
        lo_cnt = max_cpw - 2
        pltpu.sync_copy(idx_hbm.at[pl.ds(base_al, max_cpw + 8)], idx_v)
    else:
        base = wid * NUM_CHUNKS // NUM_WORKERS
        cnt = (wid + 1) * NUM_CHUNKS // NUM_WORKERS - base
        off = 0
        lo_cnt = max_cpw - 1
        pltpu.sync_copy(idx_hbm.at[pl.ds(base, max_cpw)], idx_v)

    def issue(k, b):
        pltpu.async_copy(table_hbm.at[idx_v.at[k + off]], rows[b], sems[b])

    def drain(k, b):
        pltpu.make_async_copy(table_hbm.at[idx_v.at[k + off]], rows[b],
                              sems[b]).wait()

    def reduce_chunk(k, b):
        for n in range(CHUNK):
            def red(j, acc):
                new = acc
                for jj in range(4):
                    row = n * DEGREE + j * 4 + jj
                    new = tuple(
                        new[v] + rows[b][row, pl.ds(v * LANES, LANES)]
                        for v in range(nv)
                    )
                return new

            zero = jnp.zeros((LANES,), jnp.float32)
            acc = lax.fori_loop(0, DEGREE // 4, red, (zero,) * nv)
            out_row = k * CHUNK + n
            for v in range(nv):
                out_v[out_row, pl.ds(v * LANES, LANES)] = acc[v]

    for b in range(NBUF):
        issue(b, b)

    def pair_body(i, carry):
        for b in range(NBUF):
            k = i * NBUF + b

            @pl.when(k < cnt)
            def _():
                drain(k, b)
                reduce_chunk(k, b)

                @pl.when(k + NBUF < cnt)
                def _():
                    issue(k + NBUF, b)

        return carry

    lax.fori_loop(0, -(-max_cpw // NBUF), pair_body, 0)

    row0 = base * CHUNK

    @pl.when(cnt == max_cpw)
    def _():
        pltpu.sync_copy(out_v, out_hbm.at[pl.ds(row0, max_cpw * CHUNK)])

    @pl.when(cnt == lo_cnt)
    def _():
        small = lo_cnt * CHUNK
        pltpu.sync_copy(out_v.at[pl.ds(0, small)],
                        out_hbm.at[pl.ds(row0, small)])


def _make_sc_kernel(width, aligned):
    body = functools.partial(_gather_sum_body, width=width, aligned=aligned)
    mesh = plsc.VectorSubcoreMesh(core_axis_name="c", subcore_axis_name="s")
    max_cpw = MAX_CPW_N if aligned else MAX_CPW_E
    idx_rows = max_cpw + 8 if aligned else max_cpw
    return pl.kernel(
        body,
        mesh=mesh,
        compiler_params=pltpu.CompilerParams(use_tc_tiling_on_sc=aligned),
        out_type=jax.ShapeDtypeStruct((N_NODES, width), jnp.float32),
        scratch_types=[
            pltpu.VMEM((idx_rows, IDX_PER_CHUNK), jnp.int32),
            [pltpu.VMEM((IDX_PER_CHUNK, width), jnp.float32)
             for _ in range(NBUF)],
            pltpu.VMEM((max_cpw * CHUNK, width), jnp.float32),
            [pltpu.SemaphoreType.DMA for _ in range(NBUF)],
        ],
    )


@jax.jit
def _sc_gather_sums(node_repr, edge_repr, nn2d, en2d):
    nsum = _make_sc_kernel(NODE_SIZE, True)(node_repr, nn2d)
    esum = _make_sc_kernel(EDGE_SIZE, False)(edge_repr, en2d)
    return nsum, esum


def _tc_body(nsum_ref, esum_ref, node_ref, wdn_ref, wde_ref, ws_ref, bias_ref,
             out_ref):
    act = jnp.dot(nsum_ref[:], wdn_ref[:], preferred_element_type=jnp.float32)
    act = act + jnp.dot(esum_ref[:], wde_ref[:],
                        preferred_element_type=jnp.float32)
    act = act + jnp.dot(node_ref[:], ws_ref[:],
                        preferred_element_type=jnp.float32)
    act = act + bias_ref[:]
    mean = jnp.mean(act, axis=0, keepdims=True)
    cent = act - mean
    var = jnp.mean(cent * cent, axis=0, keepdims=True)
    out_ref[:] = jnp.maximum(cent * lax.rsqrt(var + EPS), 0.0)


def _tc_finish(nsum, esum, node_repr, wdn, wde, ws, bias):
    return pl.pallas_call(
        _tc_body,
        out_shape=jax.ShapeDtypeStruct((N_NODES, OUT_SIZE), jnp.float32),
    )(nsum, esum, node_repr, wdn, wde, ws, bias)


def kernel(node_repr, edge_repr, node_neighbor, edge_neighbor, W_deg, W_self,
           bias):
    nn2d = jnp.pad(node_neighbor.reshape(NUM_CHUNKS, IDX_PER_CHUNK),
                   ((0, IDX_PAD_ROWS - NUM_CHUNKS), (0, 0)))
    en2d = edge_neighbor.reshape(NUM_CHUNKS, IDX_PER_CHUNK)
    nsum, esum = _sc_gather_sums(node_repr, edge_repr, nn2d, en2d)
    return _tc_finish(nsum, esum, node_repr, W_deg[:NODE_SIZE],
                      W_deg[NODE_SIZE:], W_self, bias)


# speedup vs baseline: 1.2740x; 1.2740x over previous
import functools

import jax
import jax.numpy as jnp
from jax import lax
from jax.experimental import pallas as pl
from jax.experimental.pallas import tpu as pltpu
from jax.experimental.pallas import tpu_sc as plsc

N_NODES = 10000
N_EDGES = 320000
DEGREE = 32
NODE_SIZE = 128
EDGE_SIZE = 16
OUT_SIZE = 128
EPS = 1e-5

NUM_WORKERS = 32
LANES = 16
IDX_ROW = 128
IDX_ROWS_TOTAL = N_NODES * DEGREE // IDX_ROW
IDX_PAD_ROWS = 2504
NBUF = 2

CHUNK_N = 8
NCH_N = N_NODES // CHUNK_N
MAX_CPW_N = -(-NCH_N // NUM_WORKERS)
CHUNK_E = 16
NCH_E = N_NODES // CHUNK_E
MAX_CPW_E = -(-NCH_E // NUM_WORKERS)


def _gather_sum_body(table_hbm, idx_hbm, out_hbm, idx_v, rows, out_v, sems,
                     *, width, chunk, max_cpw, aligned):
    nv = width // LANES
    rows_per_chunk = chunk * DEGREE // IDX_ROW
    cid = lax.axis_index("c")
    sid = lax.axis_index("s")
    wid = sid * 2 + cid
    base = wid * (N_NODES // chunk) // NUM_WORKERS
    cnt = (wid + 1) * (N_NODES // chunk) // NUM_WORKERS - base
    irow0 = base * rows_per_chunk
    n_irows = max_cpw * rows_per_chunk

    if aligned:
        al = (irow0 // 8) * 8
        off = irow0 - al
        pltpu.sync_copy(idx_hbm.at[pl.ds(al, n_irows + 8)], idx_v)
    else:
        off = 0
        pltpu.sync_copy(idx_hbm.at[pl.ds(irow0, n_irows)], idx_v)

    def issue(k, b):
        for j in range(rows_per_chunk):
            pltpu.async_copy(
                table_hbm.at[idx_v.at[k * rows_per_chunk + j + off]],
                rows[b].at[pl.ds(j * IDX_ROW, IDX_ROW)], sems[b])

    def drain(k, b):
        for j in range(rows_per_chunk):
            pltpu.make_async_copy(
                table_hbm.at[idx_v.at[k * rows_per_chunk + j + off]],
                rows[b].at[pl.ds(j * IDX_ROW, IDX_ROW)], sems[b]).wait()

    def reduce_chunk(k, b):
        for n in range(chunk):
            def red(j, acc):
                new = acc
                for jj in range(4):
                    row = n * DEGREE + j * 4 + jj
                    new = tuple(
                        new[v] + rows[b][row, pl.ds(v * LANES, LANES)]
                        for v in range(nv)
                    )
                return new

            zero = jnp.zeros((LANES,), jnp.float32)
            acc = lax.fori_loop(0, DEGREE // 4, red, (zero,) * nv)
            out_row = k * chunk + n
            for v in range(nv):
                out_v[out_row, pl.ds(v * LANES, LANES)] = acc[v]

    for b in range(NBUF):
        issue(b, b)

    def pair_body(i, carry):
        for b in range(NBUF):
            k = i * NBUF + b

            @pl.when(k < cnt)
            def _():
                drain(k, b)
                reduce_chunk(k, b)

                @pl.when(k + NBUF < cnt)
                def _():
                    issue(k + NBUF, b)

        return carry

    lax.fori_loop(0, -(-max_cpw // NBUF), pair_body, 0)

    row0 = base * chunk

    @pl.when(cnt == max_cpw)
    def _():
        pltpu.sync_copy(out_v, out_hbm.at[pl.ds(row0, max_cpw * chunk)])

    @pl.when(cnt == max_cpw - 1)
    def _():
        small = (max_cpw - 1) * chunk
        pltpu.sync_copy(out_v.at[pl.ds(0, small)],
                        out_hbm.at[pl.ds(row0, small)])


def _make_sc_kernel(width, chunk, max_cpw, aligned, n_dummy=0):
    def body(*refs):
        _gather_sum_body(*refs[:2], *refs[2 + n_dummy:], width=width,
                         chunk=chunk, max_cpw=max_cpw, aligned=aligned)

    mesh = plsc.VectorSubcoreMesh(core_axis_name="c", subcore_axis_name="s")
    rows_per_chunk = chunk * DEGREE // IDX_ROW
    idx_rows = max_cpw * rows_per_chunk + (8 if aligned else 0)
    return pl.kernel(
        body,
        mesh=mesh,
        compiler_params=pltpu.CompilerParams(use_tc_tiling_on_sc=aligned),
        out_type=jax.ShapeDtypeStruct((N_NODES, width), jnp.float32),
        scratch_types=[
            pltpu.VMEM((idx_rows, IDX_ROW), jnp.int32),
            [pltpu.VMEM((rows_per_chunk * IDX_ROW, width), jnp.float32)
             for _ in range(NBUF)],
            pltpu.VMEM((max_cpw * chunk, width), jnp.float32),
            [pltpu.SemaphoreType.DMA for _ in range(NBUF)],
        ],
    )


@jax.jit
def _sc_gather_sums(node_repr, edge_repr, nn2d, en2d):
    nsum = _make_sc_kernel(NODE_SIZE, CHUNK_N, MAX_CPW_N, True)(
        node_repr, nn2d)
    esum = _make_sc_kernel(EDGE_SIZE, CHUNK_E, MAX_CPW_E, False, n_dummy=1)(
        edge_repr, en2d, nsum)
    return nsum, esum


def _tc_body(nsum_ref, esum_ref, node_ref, wdn_ref, wde_ref, ws_ref, bias_ref,
             out_ref):
    act = jnp.dot(nsum_ref[:], wdn_ref[:], preferred_element_type=jnp.float32)
    act = act + jnp.dot(esum_ref[:], wde_ref[:],
                        preferred_element_type=jnp.float32)
    act = act + jnp.dot(node_ref[:], ws_ref[:],
                        preferred_element_type=jnp.float32)
    act = act + bias_ref[:]
    mean = jnp.mean(act, axis=0, keepdims=True)
    cent = act - mean
    var = jnp.mean(cent * cent, axis=0, keepdims=True)
    out_ref[:] = jnp.maximum(cent * lax.rsqrt(var + EPS), 0.0)


def _tc_finish(nsum, esum, node_repr, wdn, wde, ws, bias):
    return pl.pallas_call(
        _tc_body,
        out_shape=jax.ShapeDtypeStruct((N_NODES, OUT_SIZE), jnp.float32),
    )(nsum, esum, node_repr, wdn, wde, ws, bias)


def kernel(node_repr, edge_repr, node_neighbor, edge_neighbor, W_deg, W_self,
           bias):
    nn2d = jnp.pad(node_neighbor.reshape(IDX_ROWS_TOTAL, IDX_ROW),
                   ((0, IDX_PAD_ROWS - IDX_ROWS_TOTAL), (0, 0)))
    en2d = edge_neighbor.reshape(IDX_ROWS_TOTAL, IDX_ROW)
    nsum, esum = _sc_gather_sums(node_repr, edge_repr, nn2d, en2d)
    return _tc_finish(nsum, esum, node_repr, W_deg[:NODE_SIZE],
                      W_deg[NODE_SIZE:], W_self, bias)
